# baseline (speedup 1.0000x reference)
import functools

import jax
import jax.numpy as jnp
from jax import lax
from jax.experimental import pallas as pl
from jax.experimental.pallas import tpu as pltpu
from jax.experimental.pallas import tpu_sc as plsc

N_SEEDS = 1024
_CHUNK = 80
_NBI = 6
_NBR = 4
_GROUP = 12


def _spmm_pass(x, svec, row, col, zeros2, zeros1, svec_is_ones):
    N, D = x.shape
    E = row.shape[0]
    info = plsc.get_sparse_core_info()
    NC, NS = info.num_cores, info.num_subcores
    W = NC * NS
    n_chunks = E // _CHUNK
    cpw = n_chunks // W
    rem = n_chunks % W
    cpw_max = cpw + (1 if rem else 0)
    RPT = N // NS

    mesh = plsc.VectorSubcoreMesh(core_axis_name="c", subcore_axis_name="s")

    @functools.partial(
        pl.kernel,
        out_type=(
            jax.ShapeDtypeStruct((NC, N, D), jnp.float32),
            jax.ShapeDtypeStruct((NC, N), jnp.float32),
        ),
        mesh=mesh,
        scratch_types=(
            pltpu.VMEM_SHARED((N, D), jnp.float32),
            pltpu.VMEM_SHARED((N,), jnp.float32),
            [pltpu.VMEM((_CHUNK,), jnp.int32) for _ in range(_NBI)],
            [pltpu.VMEM((_CHUNK,), jnp.int32) for _ in range(_NBI)],
            [pltpu.VMEM((_CHUNK, D), jnp.float32) for _ in range(_NBR)],
            [pltpu.VMEM((_CHUNK,), jnp.float32) for _ in range(_NBR)],
            pltpu.SemaphoreType.DMA((2 * _NBI + 4 * _NBR,)),
        ),
    )
    def run(x_hbm, s_hbm, row_hbm, col_hbm, z2_hbm, z1_hbm,
            outp_hbm, outs_hbm, acc, sacc, row_v, col_v, rows_v, sv_v, sem):
        c = lax.axis_index("c")
        s = lax.axis_index("s")
        wid = c * NS + s

        pltpu.sync_copy(z2_hbm.at[pl.ds(s * RPT, RPT)], acc.at[pl.ds(s * RPT, RPT)])

        @pl.when(s == 0)
        def _():
            pltpu.sync_copy(z1_hbm, sacc)

        if svec_is_ones:
            for k in range(_NBR):
                pltpu.sync_copy(s_hbm.at[pl.ds(0, _CHUNK)], sv_v[k])

        plsc.subcore_barrier()

        start_chunk = wid * cpw + jnp.minimum(wid, rem)
        n_mine = cpw + jnp.where(wid < rem, 1, 0)

        def idx_start(j, bi):
            base = (start_chunk + j) * _CHUNK
            pltpu.make_async_copy(row_hbm.at[pl.ds(base, _CHUNK)],
                                  row_v[bi], sem.at[bi]).start()
            pltpu.make_async_copy(col_hbm.at[pl.ds(base, _CHUNK)],
                                  col_v[bi], sem.at[_NBI + bi]).start()

        def idx_wait(j, bi):
            base = (start_chunk + j) * _CHUNK
            pltpu.make_async_copy(row_hbm.at[pl.ds(base, _CHUNK)],
                                  row_v[bi], sem.at[bi]).wait()
            pltpu.make_async_copy(col_hbm.at[pl.ds(base, _CHUNK)],
                                  col_v[bi], sem.at[_NBI + bi]).wait()

        def gat_start(bi, br):
            pltpu.make_async_copy(x_hbm.at[col_v[bi]], rows_v[br],
                                  sem.at[2 * _NBI + br]).start()
            if not svec_is_ones:
                pltpu.make_async_copy(s_hbm.at[col_v[bi]], sv_v[br],
                                      sem.at[2 * _NBI + _NBR + br]).start()

        def gat_wait(bi, br):
            pltpu.make_async_copy(x_hbm.at[col_v[bi]], rows_v[br],
                                  sem.at[2 * _NBI + br]).wait()
            if not svec_is_ones:
                pltpu.make_async_copy(s_hbm.at[col_v[bi]], sv_v[br],
                                      sem.at[2 * _NBI + _NBR + br]).wait()

        def scat_start(bi, br):
            pltpu.async_copy(rows_v[br], acc.at[row_v[bi]],
                             sem.at[2 * _NBI + 2 * _NBR + br], add=True)
            pltpu.async_copy(sv_v[br], sacc.at[row_v[bi]],
                             sem.at[2 * _NBI + 3 * _NBR + br], add=True)

        def scat_wait(bi, br):
            pltpu.make_async_copy(rows_v[br], acc.at[row_v[bi]],
                                  sem.at[2 * _NBI + 2 * _NBR + br]).wait()
            pltpu.make_async_copy(sv_v[br], sacc.at[row_v[bi]],
                                  sem.at[2 * _NBI + 3 * _NBR + br]).wait()

        for k in range(4):
            @pl.when(k < n_mine)
            def _():
                idx_start(k, k)

        for k in range(2):
            @pl.when(k < n_mine)
            def _():
                idx_wait(k, k)
                gat_start(k, k)

        def group_body(g, carry):
            for u in range(_GROUP):
                j = g * _GROUP + u

                @pl.when(j < n_mine)
                def _():
                    @pl.when(j >= 2)
                    def _():
                        scat_wait((u - 2) % _NBI, (u - 2) % _NBR)

                    @pl.when(j + 4 < n_mine)
                    def _():
                        idx_start(j + 4, (u + 4) % _NBI)

                    @pl.when(j + 2 < n_mine)
                    def _():
                        idx_wait(j + 2, (u + 2) % _NBI)
                        gat_start((u + 2) % _NBI, (u + 2) % _NBR)

                    gat_wait(u % _NBI, u % _NBR)

                    @pl.when(j + 2 < n_mine)
                    def _():
                        scat_start(u % _NBI, u % _NBR)

                    @pl.when(j + 2 >= n_mine)
                    def _():
                        pltpu.sync_copy(rows_v[u % _NBR],
                                        acc.at[row_v[u % _NBI]], add=True)
                        pltpu.sync_copy(sv_v[u % _NBR],
                                        sacc.at[row_v[u % _NBI]], add=True)

            return carry

        lax.fori_loop(0, (cpw_max + _GROUP - 1) // _GROUP, group_body, 0)
        plsc.subcore_barrier()

        pltpu.sync_copy(acc.at[pl.ds(s * RPT, RPT)],
                        outp_hbm.at[c, pl.ds(s * RPT, RPT)])

        @pl.when(s == 0)
        def _():
            pltpu.sync_copy(sacc, outs_hbm.at[c])

    return run(x, svec, row, col, zeros2, zeros1)


def _first_body(p_ref, x_ref, d_ref, first_ref, deg_ref):
    first_ref[...] = (p_ref[0] + p_ref[1]) - x_ref[...]
    deg_ref[...] = d_ref[0] + d_ref[1]


def _score_body(q_ref, first_ref, x_ref, deg_ref, sden_ref, noise_ref, out_ref):
    q = q_ref[0] + q_ref[1]
    first = first_ref[...]
    x = x_ref[...]
    deg = deg_ref[...].reshape(-1, 1)
    second = q - first - deg * x
    a_deg = (sden_ref[0] + sden_ref[1]).reshape(-1, 1)
    second_num = a_deg - deg - deg
    sub = (first + second) / (deg + second_num + 1e-08)
    sn = sub / jnp.maximum(jnp.sqrt(jnp.sum(sub * sub, axis=-1, keepdims=True)), 1e-12)
    xn = x / jnp.maximum(jnp.sqrt(jnp.sum(x * x, axis=-1, keepdims=True)), 1e-12)
    dot = jnp.sum(sn * xn, axis=-1)
    out_ref[...] = jnp.log(jax.nn.sigmoid(dot)) + noise_ref[...]


def kernel(all_embeddings, edge_index, edge_weight):
    N, D = all_embeddings.shape
    E = edge_index.shape[1]

    NP = ((N + 127) // 128) * 128
    if NP == N:
        NP += 128

    EP = ((E + _CHUNK - 1) // _CHUNK) * _CHUNK
    pad = jnp.full((EP - E,), NP - 1, jnp.int32)
    row = jnp.concatenate([edge_index[0].astype(jnp.int32), pad])
    col = jnp.concatenate([edge_index[1].astype(jnp.int32), pad])

    x_p = jnp.pad(all_embeddings, ((0, NP - N), (0, 0)))

    zeros2 = jnp.zeros((NP, D), jnp.float32)
    zeros1 = jnp.zeros((NP,), jnp.float32)
    ones1 = jnp.ones((NP,), jnp.float32)

    p_part, deg_part = _spmm_pass(x_p, ones1, row, col, zeros2, zeros1, True)

    first, deg = pl.pallas_call(
        _first_body,
        out_shape=(
            jax.ShapeDtypeStruct((NP, D), jnp.float32),
            jax.ShapeDtypeStruct((NP,), jnp.float32),
        ),
    )(p_part, x_p, deg_part)

    q_part, sden_part = _spmm_pass(first, deg, row, col, zeros2, zeros1, False)

    noise = jax.random.uniform(jax.random.key(1), (N,), minval=1e-06, maxval=1.0)
    noise = -jnp.log(-jnp.log(noise))
    noise_p = jnp.pad(noise, (0, NP - N))

    scores_p = pl.pallas_call(
        _score_body,
        out_shape=jax.ShapeDtypeStruct((NP,), jnp.float32),
    )(q_part, first, x_p, deg, sden_part, noise_p)

    scores = scores_p[:N]
    _, seeds = jax.lax.top_k(scores, N_SEEDS)
    return (scores, seeds)


# speedup vs baseline: 1.0024x; 1.0024x over previous
import functools

import jax
import jax.numpy as jnp
from jax import lax
from jax.experimental import pallas as pl
from jax.experimental.pallas import tpu as pltpu
from jax.experimental.pallas import tpu_sc as plsc

N_SEEDS = 1024
_CHUNK = 80
_NBI = 6
_NBR = 4
_GROUP = 12


def _spmm_pass(x, svec, row, col, zeros2, zeros1, svec_is_ones):
    N, D = x.shape
    E = row.shape[0]
    info = plsc.get_sparse_core_info()
    NC, NS = info.num_cores, info.num_subcores
    W = NC * NS
    n_chunks = E // _CHUNK
    cpw = n_chunks // W
    rem = n_chunks % W
    cpw_max = cpw + (1 if rem else 0)
    RPT = N // NS

    mesh = plsc.VectorSubcoreMesh(core_axis_name="c", subcore_axis_name="s")

    @functools.partial(
        pl.kernel,
        out_type=(
            jax.ShapeDtypeStruct((NC, N, D), jnp.float32),
            jax.ShapeDtypeStruct((NC, N), jnp.float32),
        ),
        mesh=mesh,
        scratch_types=(
            pltpu.VMEM_SHARED((N, D), jnp.float32),
            pltpu.VMEM_SHARED((N,), jnp.float32),
            [pltpu.VMEM((_CHUNK,), jnp.int32) for _ in range(_NBI)],
            [pltpu.VMEM((_CHUNK,), jnp.int32) for _ in range(_NBI)],
            [pltpu.VMEM((_CHUNK, D), jnp.float32) for _ in range(_NBR)],
            [pltpu.VMEM((_CHUNK,), jnp.float32) for _ in range(_NBR)],
            pltpu.SemaphoreType.DMA((2 * _NBI + 4 * _NBR,)),
        ),
    )
    def run(x_hbm, s_hbm, row_hbm, col_hbm, z2_hbm, z1_hbm,
            outp_hbm, outs_hbm, acc, sacc, row_v, col_v, rows_v, sv_v, sem):
        c = lax.axis_index("c")
        s = lax.axis_index("s")
        wid = c * NS + s

        pltpu.sync_copy(z2_hbm.at[pl.ds(s * RPT, RPT)], acc.at[pl.ds(s * RPT, RPT)])

        @pl.when(s == 0)
        def _():
            pltpu.sync_copy(z1_hbm, sacc)

        if svec_is_ones:
            for k in range(_NBR):
                pltpu.sync_copy(s_hbm.at[pl.ds(0, _CHUNK)], sv_v[k])

        plsc.subcore_barrier()

        start_chunk = wid * cpw + jnp.minimum(wid, rem)
        n_mine = cpw + jnp.where(wid < rem, 1, 0)

        def idx_start(j, bi):
            base = (start_chunk + j) * _CHUNK
            pltpu.make_async_copy(row_hbm.at[pl.ds(base, _CHUNK)],
                                  row_v[bi], sem.at[bi]).start()
            pltpu.make_async_copy(col_hbm.at[pl.ds(base, _CHUNK)],
                                  col_v[bi], sem.at[_NBI + bi]).start()

        def idx_wait(j, bi):
            base = (start_chunk + j) * _CHUNK
            pltpu.make_async_copy(row_hbm.at[pl.ds(base, _CHUNK)],
                                  row_v[bi], sem.at[bi]).wait()
            pltpu.make_async_copy(col_hbm.at[pl.ds(base, _CHUNK)],
                                  col_v[bi], sem.at[_NBI + bi]).wait()

        def gat_start(bi, br):
            pltpu.make_async_copy(x_hbm.at[col_v[bi]], rows_v[br],
                                  sem.at[2 * _NBI + br]).start()
            if not svec_is_ones:
                pltpu.make_async_copy(s_hbm.at[col_v[bi]], sv_v[br],
                                      sem.at[2 * _NBI + _NBR + br]).start()

        def gat_wait(bi, br):
            pltpu.make_async_copy(x_hbm.at[col_v[bi]], rows_v[br],
                                  sem.at[2 * _NBI + br]).wait()
            if not svec_is_ones:
                pltpu.make_async_copy(s_hbm.at[col_v[bi]], sv_v[br],
                                      sem.at[2 * _NBI + _NBR + br]).wait()

        def scat_start(bi, br):
            pltpu.async_copy(rows_v[br], acc.at[row_v[bi]],
                             sem.at[2 * _NBI + 2 * _NBR + br], add=True)
            pltpu.async_copy(sv_v[br], sacc.at[row_v[bi]],
                             sem.at[2 * _NBI + 3 * _NBR + br], add=True)

        def scat_wait(bi, br):
            pltpu.make_async_copy(rows_v[br], acc.at[row_v[bi]],
                                  sem.at[2 * _NBI + 2 * _NBR + br]).wait()
            pltpu.make_async_copy(sv_v[br], sacc.at[row_v[bi]],
                                  sem.at[2 * _NBI + 3 * _NBR + br]).wait()

        for k in range(4):
            @pl.when(k < n_mine)
            def _():
                idx_start(k, k)

        for k in range(2):
            @pl.when(k < n_mine)
            def _():
                idx_wait(k, k)
                gat_start(k, k)

        def group_body(g, carry):
            for u in range(_GROUP):
                j = g * _GROUP + u

                @pl.when(j < n_mine)
                def _():
                    @pl.when(j >= 2)
                    def _():
                        scat_wait((u - 2) % _NBI, (u - 2) % _NBR)

                    @pl.when(j + 4 < n_mine)
                    def _():
                        idx_start(j + 4, (u + 4) % _NBI)

                    @pl.when(j + 2 < n_mine)
                    def _():
                        idx_wait(j + 2, (u + 2) % _NBI)
                        gat_start((u + 2) % _NBI, (u + 2) % _NBR)

                    gat_wait(u % _NBI, u % _NBR)

                    @pl.when(j + 2 < n_mine)
                    def _():
                        scat_start(u % _NBI, u % _NBR)

                    @pl.when(j + 2 >= n_mine)
                    def _():
                        pltpu.sync_copy(rows_v[u % _NBR],
                                        acc.at[row_v[u % _NBI]], add=True)
                        pltpu.sync_copy(sv_v[u % _NBR],
                                        sacc.at[row_v[u % _NBI]], add=True)

            return carry

        lax.fori_loop(0, (cpw_max + _GROUP - 1) // _GROUP, group_body, 0)
        plsc.subcore_barrier()

        pltpu.sync_copy(acc.at[pl.ds(s * RPT, RPT)],
                        outp_hbm.at[c, pl.ds(s * RPT, RPT)])

        @pl.when(s == 0)
        def _():
            pltpu.sync_copy(sacc, outs_hbm.at[c])

    return run(x, svec, row, col, zeros2, zeros1)


def _first_body(p_ref, x_ref, d_ref, first_ref, deg_ref):
    first_ref[...] = (p_ref[0] + p_ref[1]) - x_ref[...]
    deg_ref[...] = d_ref[0] + d_ref[1]


def _xn_body(x_ref, xn_ref):
    x = x_ref[...]
    xn_ref[...] = x / jnp.maximum(
        jnp.sqrt(jnp.sum(x * x, axis=-1, keepdims=True)), 1e-12)


def _score_body(q_ref, first_ref, x_ref, xn_ref, deg_ref, sden_ref, noise_ref,
                out_ref):
    q = q_ref[0] + q_ref[1]
    first = first_ref[...]
    x = x_ref[...]
    deg = deg_ref[...].reshape(-1, 1)
    second = q - first - deg * x
    a_deg = (sden_ref[0] + sden_ref[1]).reshape(-1, 1)
    second_num = a_deg - deg - deg
    sub = (first + second) / (deg + second_num + 1e-08)
    sn = sub / jnp.maximum(jnp.sqrt(jnp.sum(sub * sub, axis=-1, keepdims=True)), 1e-12)
    dot = jnp.sum(sn * xn_ref[...], axis=-1)
    out_ref[...] = jnp.log(jax.nn.sigmoid(dot)) + noise_ref[...]


def kernel(all_embeddings, edge_index, edge_weight):
    N, D = all_embeddings.shape
    E = edge_index.shape[1]

    NP = ((N + 127) // 128) * 128
    if NP == N:
        NP += 128

    EP = ((E + _CHUNK - 1) // _CHUNK) * _CHUNK
    pad = jnp.full((EP - E,), NP - 1, jnp.int32)
    row = jnp.concatenate([edge_index[0].astype(jnp.int32), pad])
    col = jnp.concatenate([edge_index[1].astype(jnp.int32), pad])

    x_p = jnp.pad(all_embeddings, ((0, NP - N), (0, 0)))

    zeros2 = jnp.zeros((NP, D), jnp.float32)
    zeros1 = jnp.zeros((NP,), jnp.float32)
    ones1 = jnp.ones((NP,), jnp.float32)

    p_part, deg_part = _spmm_pass(x_p, ones1, row, col, zeros2, zeros1, True)

    first, deg = pl.pallas_call(
        _first_body,
        out_shape=(
            jax.ShapeDtypeStruct((NP, D), jnp.float32),
            jax.ShapeDtypeStruct((NP,), jnp.float32),
        ),
    )(p_part, x_p, deg_part)

    xn = pl.pallas_call(
        _xn_body,
        out_shape=jax.ShapeDtypeStruct((NP, D), jnp.float32),
    )(x_p)

    q_part, sden_part = _spmm_pass(first, deg, row, col, zeros2, zeros1, False)

    noise = jax.random.uniform(jax.random.key(1), (N,), minval=1e-06, maxval=1.0)
    noise = -jnp.log(-jnp.log(noise))
    noise_p = jnp.pad(noise, (0, NP - N))

    scores_p = pl.pallas_call(
        _score_body,
        out_shape=jax.ShapeDtypeStruct((NP,), jnp.float32),
    )(q_part, first, x_p, xn, deg, sden_part, noise_p)

    scores = scores_p[:N]
    _, seeds = jax.lax.top_k(scores, N_SEEDS)
    return (scores, seeds)
